# baseline (speedup 1.0000x reference)
import functools

import jax
import jax.numpy as jnp
from jax.experimental import pallas as pl
from jax.experimental.pallas import tpu as pltpu

_B, _N, _DIM = 1, 2048, 768
_E = 8
_HID = 4 * _DIM
_CAP = 512
_HC = 1536
_NSTEPS = _HID // _HC


def _ffn_kernel(idx_ref, x_ref, w1_ref, w2_ref, out_ref):
    del idx_ref
    step = pl.program_id(0)

    h = jnp.dot(x_ref[...], w1_ref[0], preferred_element_type=jnp.float32)
    h = 0.5 * h * (1.0 + jax.lax.erf(h * 0.7071067811865476))
    y = jnp.dot(h, w2_ref[0], preferred_element_type=jnp.float32)

    @pl.when(step == 0)
    def _init():
        out_ref[...] = jnp.zeros_like(out_ref)

    out_ref[pl.ds(0, _CAP), :] += y


@jax.jit
def kernel(inputs, current_y, w1, w2):
    x2d = inputs.reshape(_N, _DIM)
    exp_idx = jnp.remainder(current_y, _E).astype(jnp.int32)

    grid_spec = pltpu.PrefetchScalarGridSpec(
        num_scalar_prefetch=1,
        grid=(_NSTEPS,),
        in_specs=[
            pl.BlockSpec((_CAP, _DIM), lambda i, idx: (0, 0)),
            pl.BlockSpec((1, _DIM, _HC), lambda i, idx: (idx[0], 0, i)),
            pl.BlockSpec((1, _HC, _DIM), lambda i, idx: (idx[0], i, 0)),
        ],
        out_specs=pl.BlockSpec((_N, _DIM), lambda i, idx: (0, 0)),
    )

    out2d = pl.pallas_call(
        _ffn_kernel,
        grid_spec=grid_spec,
        out_shape=jax.ShapeDtypeStruct((_N, _DIM), jnp.float32),
    )(exp_idx, x2d, w1, w2)

    return out2d.reshape(_B, _N, _DIM)


# speedup vs baseline: 1.0194x; 1.0194x over previous
import functools

import jax
import jax.numpy as jnp
from jax.experimental import pallas as pl
from jax.experimental.pallas import tpu as pltpu

_B, _N, _DIM = 1, 2048, 768
_E = 8
_HID = 4 * _DIM
_CAP = 512
_HC = 1024
_NSTEPS = _HID // _HC


def _ffn_kernel(idx_ref, x_ref, w1_ref, w2_ref, out_ref):
    del idx_ref
    step = pl.program_id(0)

    h = jnp.dot(x_ref[...], w1_ref[0], preferred_element_type=jnp.float32)
    h = 0.5 * h * (1.0 + jax.lax.erf(h * 0.7071067811865476))
    y = jnp.dot(h, w2_ref[0], preferred_element_type=jnp.float32)

    @pl.when(step == 0)
    def _init():
        out_ref[...] = jnp.zeros_like(out_ref)

    out_ref[pl.ds(0, _CAP), :] += y


@jax.jit
def kernel(inputs, current_y, w1, w2):
    x2d = inputs.reshape(_N, _DIM)
    exp_idx = current_y.astype(jnp.int32)

    grid_spec = pltpu.PrefetchScalarGridSpec(
        num_scalar_prefetch=1,
        grid=(_NSTEPS,),
        in_specs=[
            pl.BlockSpec((_CAP, _DIM), lambda i, idx: (0, 0)),
            pl.BlockSpec((1, _DIM, _HC), lambda i, idx: (idx[0] % _E, 0, i)),
            pl.BlockSpec((1, _HC, _DIM), lambda i, idx: (idx[0] % _E, i, 0)),
        ],
        out_specs=pl.BlockSpec((_N, _DIM), lambda i, idx: (0, 0)),
    )

    out2d = pl.pallas_call(
        _ffn_kernel,
        grid_spec=grid_spec,
        out_shape=jax.ShapeDtypeStruct((_N, _DIM), jnp.float32),
    )(exp_idx, x2d, w1, w2)

    return out2d.reshape(_B, _N, _DIM)
